# baseline (speedup 1.0000x reference)
import jax
import jax.numpy as jnp
from jax import lax
from jax.experimental import pallas as pl
from jax.experimental.pallas import tpu as pltpu
from jax.experimental.pallas import tpu_sc as plsc

N_NODES = 10000
N_EDGES = 320000
D_FEAT = 128

NC = 2
NS = 16
NW = NC * NS
CHUNK = 128
CH_PER_W = 80
HALF_CH = 40
E_PAD = NW * CH_PER_W * CHUNK
ROWS_PER_TEC = 624
ROWS_TAIL = N_NODES - NS * ROWS_PER_TEC
FV = D_FEAT // 16


def _scale_chunk(g, evb, c):
    def grp_body(grp, _):
        evg = evb[c, pl.ds(grp * 16, 16)]
        for i in range(16):
            s = evg[i]
            e = grp * 16 + i
            for f in range(FV):
                g[e, pl.ds(f * 16, 16)] = g[e, pl.ds(f * 16, 16)] * s
        return _
    lax.fori_loop(0, CHUNK // 16, grp_body, None)


def _sc_body(x_hbm, rows_hbm, cols_hbm, ev_hbm, out_hbm,
             acc_sh, g0, g1, colb, rowb, evb, sem0, sem1, sems0, sems1):
    cid = lax.axis_index("c")
    sid = lax.axis_index("s")
    wid = sid * NC + cid

    def zrow(r, _):
        for f in range(FV):
            g0[r, pl.ds(f * 16, 16)] = jnp.zeros((16,), jnp.float32)
        return _
    lax.fori_loop(0, CHUNK, zrow, None)

    base = sid * ROWS_PER_TEC
    for j in range(4):
        pltpu.sync_copy(g0, acc_sh.at[pl.ds(base + j * CHUNK, CHUNK)])
    pltpu.sync_copy(g0.at[pl.ds(0, ROWS_PER_TEC - 4 * CHUNK)],
                    acc_sh.at[pl.ds(base + 4 * CHUNK, ROWS_PER_TEC - 4 * CHUNK)])

    @pl.when(sid == 0)
    def _zero_tail():
        pltpu.sync_copy(g0.at[pl.ds(0, ROWS_TAIL)],
                        acc_sh.at[pl.ds(NS * ROWS_PER_TEC, ROWS_TAIL)])
    plsc.subcore_barrier()

    for h in range(CH_PER_W // HALF_CH):
        rbase = wid * CH_PER_W + h * HALF_CH
        pltpu.sync_copy(cols_hbm.at[pl.ds(rbase, HALF_CH)], colb)
        pltpu.sync_copy(rows_hbm.at[pl.ds(rbase, HALF_CH)], rowb)
        pltpu.sync_copy(ev_hbm.at[pl.ds(rbase, HALF_CH)], evb)

        pltpu.async_copy(x_hbm.at[colb.at[0]], g0, sem0)
        pltpu.async_copy(x_hbm.at[colb.at[1]], g1, sem1)

        def pair(k, _):
            c0 = 2 * k
            c1 = c0 + 1
            pltpu.make_async_copy(x_hbm.at[colb.at[c0]], g0, sem0).wait()
            _scale_chunk(g0, evb, c0)
            pltpu.async_copy(g0, acc_sh.at[rowb.at[c0]], sems0, add=True)

            pltpu.make_async_copy(x_hbm.at[colb.at[c1]], g1, sem1).wait()
            _scale_chunk(g1, evb, c1)
            pltpu.async_copy(g1, acc_sh.at[rowb.at[c1]], sems1, add=True)

            pltpu.make_async_copy(g0, acc_sh.at[rowb.at[c0]], sems0).wait()

            @pl.when(k < HALF_CH // 2 - 1)
            def _next0():
                pltpu.async_copy(x_hbm.at[colb.at[c0 + 2]], g0, sem0)

            pltpu.make_async_copy(g1, acc_sh.at[rowb.at[c1]], sems1).wait()

            @pl.when(k < HALF_CH // 2 - 1)
            def _next1():
                pltpu.async_copy(x_hbm.at[colb.at[c1 + 2]], g1, sem1)
            return _
        lax.fori_loop(0, HALF_CH // 2, pair, None)

    plsc.subcore_barrier()
    pltpu.sync_copy(acc_sh.at[pl.ds(base, ROWS_PER_TEC)],
                    out_hbm.at[cid, pl.ds(base, ROWS_PER_TEC)])

    @pl.when(sid == 0)
    def _out_tail():
        pltpu.sync_copy(acc_sh.at[pl.ds(NS * ROWS_PER_TEC, ROWS_TAIL)],
                        out_hbm.at[cid, pl.ds(NS * ROWS_PER_TEC, ROWS_TAIL)])


@jax.jit
def _sc_spmm(x, rows2d, cols2d, ev2d):
    mesh = plsc.VectorSubcoreMesh(core_axis_name="c", subcore_axis_name="s",
                                  num_cores=NC, num_subcores=NS)
    return pl.kernel(
        _sc_body,
        out_type=jax.ShapeDtypeStruct((NC, N_NODES, D_FEAT), jnp.float32),
        mesh=mesh,
        scratch_types=[
            pltpu.VMEM_SHARED((N_NODES, D_FEAT), jnp.float32),
            pltpu.VMEM((CHUNK, D_FEAT), jnp.float32),
            pltpu.VMEM((CHUNK, D_FEAT), jnp.float32),
            pltpu.VMEM((HALF_CH, CHUNK), jnp.int32),
            pltpu.VMEM((HALF_CH, CHUNK), jnp.int32),
            pltpu.VMEM((HALF_CH, CHUNK), jnp.float32),
            pltpu.SemaphoreType.DMA,
            pltpu.SemaphoreType.DMA,
            pltpu.SemaphoreType.DMA,
            pltpu.SemaphoreType.DMA,
        ],
    )(x, rows2d, cols2d, ev2d)


def _combine_body(p_ref, w_ref, o_ref):
    o_ref[...] = jnp.tanh((p_ref[0] + p_ref[1]) * w_ref[...])


@jax.jit
def _tc_combine(partials, w0):
    blk = 2000
    return pl.pallas_call(
        _combine_body,
        grid=(N_NODES // blk,),
        in_specs=[
            pl.BlockSpec((NC, blk, D_FEAT), lambda i: (0, i, 0)),
            pl.BlockSpec((1, D_FEAT), lambda i: (0, 0)),
        ],
        out_specs=pl.BlockSpec((blk, D_FEAT), lambda i: (i, 0)),
        out_shape=jax.ShapeDtypeStruct((N_NODES, D_FEAT), jnp.float32),
    )(partials, w0)


def kernel(inlayer, edge_index, edge_values, w0):
    rows = edge_index[0].astype(jnp.int32)
    cols = edge_index[1].astype(jnp.int32)
    pad = E_PAD - N_EDGES
    spread = (jnp.arange(pad, dtype=jnp.int32) * 13) % N_NODES
    rows2d = jnp.concatenate([rows, spread]).reshape(-1, CHUNK)
    cols2d = jnp.concatenate([cols, spread]).reshape(-1, CHUNK)
    ev2d = jnp.concatenate(
        [edge_values, jnp.zeros((pad,), jnp.float32)]).reshape(-1, CHUNK)
    partials = _sc_spmm(inlayer, rows2d, cols2d, ev2d)
    return _tc_combine(partials, w0)


# speedup vs baseline: 1.1040x; 1.1040x over previous
import jax
import jax.numpy as jnp
from jax import lax
from jax.experimental import pallas as pl
from jax.experimental.pallas import tpu as pltpu
from jax.experimental.pallas import tpu_sc as plsc

N_NODES = 10000
N_EDGES = 320000
D_FEAT = 128

NC = 2
NS = 16
NW = NC * NS
CHUNK = 128
CH_PER_W = 80
HALF_CH = 40
E_PAD = NW * CH_PER_W * CHUNK
ROWS_PER_TEC = 624
ROWS_TAIL = N_NODES - NS * ROWS_PER_TEC
FV = D_FEAT // 16


def _scale_chunk(g, evb, c):
    @plsc.parallel_loop(0, CHUNK // 16, step=1, unroll=2)
    def grp_body(grp):
        evg = evb[c, pl.ds(grp * 16, 16)]
        for i in range(16):
            s = evg[i]
            e = grp * 16 + i
            for f in range(FV):
                g[e, pl.ds(f * 16, 16)] = g[e, pl.ds(f * 16, 16)] * s


def _sc_body(x_hbm, rows_hbm, cols_hbm, ev_hbm, out_hbm,
             acc_sh, g0, g1, colb, rowb, evb, sem0, sem1, sems0, sems1):
    cid = lax.axis_index("c")
    sid = lax.axis_index("s")
    wid = sid * NC + cid

    def zrow(r, _):
        for f in range(FV):
            g0[r, pl.ds(f * 16, 16)] = jnp.zeros((16,), jnp.float32)
        return _
    lax.fori_loop(0, CHUNK, zrow, None)

    base = sid * ROWS_PER_TEC
    for j in range(4):
        pltpu.sync_copy(g0, acc_sh.at[pl.ds(base + j * CHUNK, CHUNK)])
    pltpu.sync_copy(g0.at[pl.ds(0, ROWS_PER_TEC - 4 * CHUNK)],
                    acc_sh.at[pl.ds(base + 4 * CHUNK, ROWS_PER_TEC - 4 * CHUNK)])

    @pl.when(sid == 0)
    def _zero_tail():
        pltpu.sync_copy(g0.at[pl.ds(0, ROWS_TAIL)],
                        acc_sh.at[pl.ds(NS * ROWS_PER_TEC, ROWS_TAIL)])
    plsc.subcore_barrier()

    for h in range(CH_PER_W // HALF_CH):
        rbase = wid * CH_PER_W + h * HALF_CH
        pltpu.sync_copy(cols_hbm.at[pl.ds(rbase, HALF_CH)], colb)
        pltpu.sync_copy(rows_hbm.at[pl.ds(rbase, HALF_CH)], rowb)
        pltpu.sync_copy(ev_hbm.at[pl.ds(rbase, HALF_CH)], evb)

        pltpu.async_copy(x_hbm.at[colb.at[0]], g0, sem0)
        pltpu.async_copy(x_hbm.at[colb.at[1]], g1, sem1)

        def pair(k, _):
            c0 = 2 * k
            c1 = c0 + 1
            pltpu.make_async_copy(x_hbm.at[colb.at[c0]], g0, sem0).wait()
            _scale_chunk(g0, evb, c0)
            pltpu.sync_copy(g0, acc_sh.at[rowb.at[c0]], add=True)

            @pl.when(k < HALF_CH // 2 - 1)
            def _next0():
                pltpu.async_copy(x_hbm.at[colb.at[c0 + 2]], g0, sem0)

            pltpu.make_async_copy(x_hbm.at[colb.at[c1]], g1, sem1).wait()
            _scale_chunk(g1, evb, c1)
            pltpu.sync_copy(g1, acc_sh.at[rowb.at[c1]], add=True)

            @pl.when(k < HALF_CH // 2 - 1)
            def _next1():
                pltpu.async_copy(x_hbm.at[colb.at[c1 + 2]], g1, sem1)
            return _
        lax.fori_loop(0, HALF_CH // 2, pair, None)

    plsc.subcore_barrier()
    pltpu.sync_copy(acc_sh.at[pl.ds(base, ROWS_PER_TEC)],
                    out_hbm.at[cid, pl.ds(base, ROWS_PER_TEC)])

    @pl.when(sid == 0)
    def _out_tail():
        pltpu.sync_copy(acc_sh.at[pl.ds(NS * ROWS_PER_TEC, ROWS_TAIL)],
                        out_hbm.at[cid, pl.ds(NS * ROWS_PER_TEC, ROWS_TAIL)])


@jax.jit
def _sc_spmm(x, rows2d, cols2d, ev2d):
    mesh = plsc.VectorSubcoreMesh(core_axis_name="c", subcore_axis_name="s",
                                  num_cores=NC, num_subcores=NS)
    return pl.kernel(
        _sc_body,
        out_type=jax.ShapeDtypeStruct((NC, N_NODES, D_FEAT), jnp.float32),
        mesh=mesh,
        scratch_types=[
            pltpu.VMEM_SHARED((N_NODES, D_FEAT), jnp.float32),
            pltpu.VMEM((CHUNK, D_FEAT), jnp.float32),
            pltpu.VMEM((CHUNK, D_FEAT), jnp.float32),
            pltpu.VMEM((HALF_CH, CHUNK), jnp.int32),
            pltpu.VMEM((HALF_CH, CHUNK), jnp.int32),
            pltpu.VMEM((HALF_CH, CHUNK), jnp.float32),
            pltpu.SemaphoreType.DMA,
            pltpu.SemaphoreType.DMA,
            pltpu.SemaphoreType.DMA,
            pltpu.SemaphoreType.DMA,
        ],
    )(x, rows2d, cols2d, ev2d)


def _combine_body(p_ref, w_ref, o_ref):
    o_ref[...] = jnp.tanh((p_ref[0] + p_ref[1]) * w_ref[...])


@jax.jit
def _tc_combine(partials, w0):
    blk = 2000
    return pl.pallas_call(
        _combine_body,
        grid=(N_NODES // blk,),
        in_specs=[
            pl.BlockSpec((NC, blk, D_FEAT), lambda i: (0, i, 0)),
            pl.BlockSpec((1, D_FEAT), lambda i: (0, 0)),
        ],
        out_specs=pl.BlockSpec((blk, D_FEAT), lambda i: (i, 0)),
        out_shape=jax.ShapeDtypeStruct((N_NODES, D_FEAT), jnp.float32),
    )(partials, w0)


def kernel(inlayer, edge_index, edge_values, w0):
    rows = edge_index[0].astype(jnp.int32)
    cols = edge_index[1].astype(jnp.int32)
    pad = E_PAD - N_EDGES
    spread = (jnp.arange(pad, dtype=jnp.int32) * 13) % N_NODES
    rows2d = jnp.concatenate([rows, spread]).reshape(-1, CHUNK)
    cols2d = jnp.concatenate([cols, spread]).reshape(-1, CHUNK)
    ev2d = jnp.concatenate(
        [edge_values, jnp.zeros((pad,), jnp.float32)]).reshape(-1, CHUNK)
    partials = _sc_spmm(inlayer, rows2d, cols2d, ev2d)
    return _tc_combine(partials, w0)
